# baseline (speedup 1.0000x reference)
import jax
import jax.numpy as jnp
from jax import lax
from jax.experimental import pallas as pl
from jax.experimental.pallas import tpu as pltpu

N = 256
D = 128
OUTDIM = 128
BI = 8
NSTEPS = N // BI


def _pool_kernel(x_ref, a_ref, w1b_ref, b1_ref, r_ref, acc_ref):
    s = pl.program_id(0)

    @pl.when(s == 0)
    def _init():
        acc_ref[...] = jnp.zeros_like(acc_ref)

    acc_ref[...] += jnp.sum(x_ref[...], axis=0)

    @pl.when(s == NSTEPS - 1)
    def _finish():
        xp = acc_ref[...] * (1.0 / N)
        y = lax.dot_general(a_ref[...], xp, (((0,), (0,)), ((), ())),
                            preferred_element_type=jnp.float32)
        r_ref[...] = lax.dot_general(
            y, w1b_ref[...], (((1,), (0,)), ((), ())),
            preferred_element_type=jnp.float32) + b1_ref[...]


def _main_kernel(x_ref, a_ref, w1a_ref, r_ref, out_ref):
    xb = x_ref[...]
    t = lax.dot_general(
        xb.reshape(BI * N, D), w1a_ref[...], (((1,), (0,)), ((), ())),
        preferred_element_type=jnp.float32)
    t = t.reshape(BI, N, OUTDIM)
    a = a_ref[...]
    for i in range(BI):
        g = lax.dot_general(a, t[i], (((0,), (0,)), ((), ())),
                            preferred_element_type=jnp.float32)
        out_ref[i] = jnp.maximum(g + r_ref[i][None, :], 0.0)


@jax.jit
def kernel(A, X, W1, b1):
    W1a = W1[:D]
    W1b = W1[D:]
    b1r = b1.reshape(1, OUTDIM)

    r = pl.pallas_call(
        _pool_kernel,
        grid=(NSTEPS,),
        in_specs=[
            pl.BlockSpec((BI, N, D), lambda s: (s, 0, 0)),
            pl.BlockSpec((N, N), lambda s: (0, 0)),
            pl.BlockSpec((D, OUTDIM), lambda s: (0, 0)),
            pl.BlockSpec((1, OUTDIM), lambda s: (0, 0)),
        ],
        out_specs=pl.BlockSpec((N, OUTDIM), lambda s: (0, 0)),
        out_shape=jax.ShapeDtypeStruct((N, OUTDIM), jnp.float32),
        scratch_shapes=[pltpu.VMEM((N, D), jnp.float32)],
    )(X, A, W1b, b1r)

    out = pl.pallas_call(
        _main_kernel,
        grid=(NSTEPS,),
        in_specs=[
            pl.BlockSpec((BI, N, D), lambda s: (s, 0, 0)),
            pl.BlockSpec((N, N), lambda s: (0, 0)),
            pl.BlockSpec((D, OUTDIM), lambda s: (0, 0)),
            pl.BlockSpec((BI, OUTDIM), lambda s: (s, 0)),
        ],
        out_specs=pl.BlockSpec((BI, N, OUTDIM), lambda s: (s, 0, 0)),
        out_shape=jax.ShapeDtypeStruct((N, N, OUTDIM), jnp.float32),
    )(X, A, W1a, r)
    return out


# speedup vs baseline: 1.2420x; 1.2420x over previous
import jax
import jax.numpy as jnp
from jax import lax
from jax.experimental import pallas as pl
from jax.experimental.pallas import tpu as pltpu

N = 256
D = 128
OUTDIM = 128
BI = 8
NSTEPS = N // BI


def _fused_kernel(x_ref, a_ref, w1a_ref, w1b_ref, b1_ref, out_ref,
                  g_ref, ysum_ref, r_ref):
    p = pl.program_id(0)
    s = pl.program_id(1)
    base = s * BI

    @pl.when(p == 0)
    def _phase0():
        @pl.when(s == 0)
        def _init():
            ysum_ref[...] = jnp.zeros_like(ysum_ref)

        a = a_ref[...]
        gs = []
        for i in range(BI):
            gs.append(lax.dot_general(a, x_ref[i], (((0,), (0,)), ((), ())),
                                      preferred_element_type=jnp.float32))
        gblk = jnp.stack(gs, axis=0)
        g_ref[pl.ds(base, BI)] = gblk
        ysum_ref[...] += jnp.sum(gblk, axis=0)

        @pl.when(s == NSTEPS - 1)
        def _finish():
            y = ysum_ref[...] * (1.0 / N)
            r_ref[...] = lax.dot_general(
                y, w1b_ref[...], (((1,), (0,)), ((), ())),
                preferred_element_type=jnp.float32) + b1_ref[...]

    @pl.when(p == 1)
    def _phase1():
        gblk = g_ref[pl.ds(base, BI)]
        r = r_ref[pl.ds(base, BI)]
        t = lax.dot_general(
            gblk.reshape(BI * N, D), w1a_ref[...], (((1,), (0,)), ((), ())),
            preferred_element_type=jnp.float32).reshape(BI, N, OUTDIM)
        out_ref[...] = jnp.maximum(t + r[:, None, :], 0.0)


@jax.jit
def kernel(A, X, W1, b1):
    W1a = W1[:D]
    W1b = W1[D:]
    b1r = b1.reshape(1, OUTDIM)

    out = pl.pallas_call(
        _fused_kernel,
        grid=(2, NSTEPS),
        in_specs=[
            pl.BlockSpec((BI, N, D), lambda p, s: ((1 - p) * s, 0, 0)),
            pl.BlockSpec((N, N), lambda p, s: (0, 0)),
            pl.BlockSpec((D, OUTDIM), lambda p, s: (0, 0)),
            pl.BlockSpec((D, OUTDIM), lambda p, s: (0, 0)),
            pl.BlockSpec((1, OUTDIM), lambda p, s: (0, 0)),
        ],
        out_specs=pl.BlockSpec((BI, N, OUTDIM), lambda p, s: (p * s, 0, 0)),
        out_shape=jax.ShapeDtypeStruct((N, N, OUTDIM), jnp.float32),
        scratch_shapes=[
            pltpu.VMEM((N, N, D), jnp.float32),
            pltpu.VMEM((N, D), jnp.float32),
            pltpu.VMEM((N, OUTDIM), jnp.float32),
        ],
    )(X, A, W1a, W1b, b1r)
    return out


# speedup vs baseline: 1.6902x; 1.3609x over previous
import jax
import jax.numpy as jnp
from jax import lax
from jax.experimental import pallas as pl
from jax.experimental.pallas import tpu as pltpu

N = 256
D = 128
OUTDIM = 128
BI = 16
NSTEPS = N // BI


def _fused_kernel(x_ref, at_ref, w1a_ref, w1b_ref, b1_ref, out_ref,
                  g_ref, ysum_ref, r_ref):
    p = pl.program_id(0)
    s = pl.program_id(1)
    base = s * BI

    @pl.when(p == 0)
    def _phase0():
        at = at_ref[...]
        acc = None
        for i in range(BI):
            gi = lax.dot_general(at, x_ref[i], (((1,), (0,)), ((), ())),
                                 preferred_element_type=jnp.float32)
            g_ref[base + i] = gi
            acc = gi if acc is None else acc + gi

        @pl.when(s == 0)
        def _init():
            ysum_ref[...] = acc

        @pl.when(s > 0)
        def _acc():
            ysum_ref[...] += acc

        @pl.when(s == NSTEPS - 1)
        def _finish():
            y = ysum_ref[...] * (1.0 / N)
            r_ref[...] = lax.dot_general(
                y, w1b_ref[...], (((1,), (0,)), ((), ())),
                preferred_element_type=jnp.float32) + b1_ref[...]

    @pl.when(p == 1)
    def _phase1():
        gblk = g_ref[pl.ds(base, BI)]
        r = r_ref[pl.ds(base, BI)]
        t = lax.dot_general(
            gblk.reshape(BI * N, D), w1a_ref[...], (((1,), (0,)), ((), ())),
            preferred_element_type=jnp.float32).reshape(BI, N, OUTDIM)
        out_ref[...] = jnp.maximum(t + r[:, None, :], 0.0)


@jax.jit
def kernel(A, X, W1, b1):
    At = A.T
    W1a = W1[:D]
    W1b = W1[D:]
    b1r = b1.reshape(1, OUTDIM)

    out = pl.pallas_call(
        _fused_kernel,
        grid=(2, NSTEPS),
        in_specs=[
            pl.BlockSpec((BI, N, D), lambda p, s: ((1 - p) * s, 0, 0)),
            pl.BlockSpec((N, N), lambda p, s: (0, 0)),
            pl.BlockSpec((D, OUTDIM), lambda p, s: (0, 0)),
            pl.BlockSpec((D, OUTDIM), lambda p, s: (0, 0)),
            pl.BlockSpec((1, OUTDIM), lambda p, s: (0, 0)),
        ],
        out_specs=pl.BlockSpec((BI, N, OUTDIM), lambda p, s: (p * s, 0, 0)),
        out_shape=jax.ShapeDtypeStruct((N, N, OUTDIM), jnp.float32),
        scratch_shapes=[
            pltpu.VMEM((N, N, D), jnp.float32),
            pltpu.VMEM((N, D), jnp.float32),
            pltpu.VMEM((N, OUTDIM), jnp.float32),
        ],
    )(X, At, W1a, W1b, b1r)
    return out


# speedup vs baseline: 2.0183x; 1.1941x over previous
import jax
import jax.numpy as jnp
from jax import lax
from jax.experimental import pallas as pl
from jax.experimental.pallas import tpu as pltpu

N = 256
D = 128
OUTDIM = 128
BI = 32
NSTEPS = N // BI


def _fused_kernel(x_ref, at_ref, w1a_ref, w1b_ref, b1_ref, out_ref,
                  g_ref, ysum_ref, r_ref):
    p = pl.program_id(0)
    s = pl.program_id(1)
    base = s * BI

    @pl.when(p == 0)
    def _phase0():
        at = at_ref[...]
        acc = None
        for i in range(BI):
            gi = lax.dot_general(at, x_ref[i], (((1,), (0,)), ((), ())),
                                 preferred_element_type=jnp.float32)
            g_ref[base + i] = gi
            acc = gi if acc is None else acc + gi

        @pl.when(s == 0)
        def _init():
            ysum_ref[...] = acc

        @pl.when(s > 0)
        def _acc():
            ysum_ref[...] += acc

        @pl.when(s == NSTEPS - 1)
        def _finish():
            y = ysum_ref[...] * (1.0 / N)
            r_ref[...] = lax.dot_general(
                y, w1b_ref[...], (((1,), (0,)), ((), ())),
                preferred_element_type=jnp.float32) + b1_ref[...]

    @pl.when(p == 1)
    def _phase1():
        gblk = g_ref[pl.ds(base, BI)]
        r = r_ref[pl.ds(base, BI)]
        t = lax.dot_general(
            gblk.reshape(BI * N, D), w1a_ref[...], (((1,), (0,)), ((), ())),
            preferred_element_type=jnp.float32).reshape(BI, N, OUTDIM)
        out_ref[...] = jnp.maximum(t + r[:, None, :], 0.0)


@jax.jit
def kernel(A, X, W1, b1):
    At = A.T
    W1a = W1[:D]
    W1b = W1[D:]
    b1r = b1.reshape(1, OUTDIM)

    out = pl.pallas_call(
        _fused_kernel,
        grid=(2, NSTEPS),
        in_specs=[
            pl.BlockSpec((BI, N, D), lambda p, s: ((1 - p) * s, 0, 0)),
            pl.BlockSpec((N, N), lambda p, s: (0, 0)),
            pl.BlockSpec((D, OUTDIM), lambda p, s: (0, 0)),
            pl.BlockSpec((D, OUTDIM), lambda p, s: (0, 0)),
            pl.BlockSpec((1, OUTDIM), lambda p, s: (0, 0)),
        ],
        out_specs=pl.BlockSpec((BI, N, OUTDIM), lambda p, s: (p * s, 0, 0)),
        out_shape=jax.ShapeDtypeStruct((N, N, OUTDIM), jnp.float32),
        scratch_shapes=[
            pltpu.VMEM((N, N, D), jnp.float32),
            pltpu.VMEM((N, D), jnp.float32),
            pltpu.VMEM((N, OUTDIM), jnp.float32),
        ],
    )(X, At, W1a, W1b, b1r)
    return out
